# baseline (speedup 1.0000x reference)
import jax
import jax.numpy as jnp
from jax import lax
from jax.experimental import pallas as pl
from jax.experimental.pallas import tpu as pltpu
from jax.experimental.pallas import tpu_sc as plsc

_NC = 2
_NS = 16
_NW = _NC * _NS
_D = 64
_TC = 10
_BPW = 32
_Q = 4
_QROWS = _BPW * _TC // _Q


def _build(B, T):
    nchunk = T // _TC
    qtot = nchunk * _Q
    rows = _BPW * _TC
    mesh = plsc.VectorSubcoreMesh(
        core_axis_name="c", subcore_axis_name="s",
        num_cores=_NC, num_subcores=_NS)

    def body(idx_hbm, tok_hbm, pos_hbm, out_hbm,
             idx_v, pos_v, g0, g1, o0, o1, gs0, gs1, os0, os1):
        w = lax.axis_index("s") * _NC + lax.axis_index("c")
        b0 = w * _BPW
        pltpu.sync_copy(idx_hbm.at[w], idx_v)
        pltpu.sync_copy(pos_hbm, pos_v)

        gbuf = (g0, g1)
        obuf = (o0, o1)
        gsem = (gs0, gs1)
        osem = (os0, os1)

        def start_gather(ct, h):
            for q in range(_Q):
                pltpu.async_copy(
                    tok_hbm.at[idx_v.at[ct * _Q + q]],
                    gbuf[h].at[pl.ds(q * _QROWS, _QROWS)], gsem[h])

        def wait_gather(ct, h):
            for q in range(_Q):
                pltpu.make_async_copy(
                    tok_hbm.at[idx_v.at[ct * _Q + q]],
                    gbuf[h].at[pl.ds(q * _QROWS, _QROWS)], gsem[h]).wait()

        def start_out(ct, h):
            pltpu.async_copy(
                obuf[h],
                out_hbm.at[pl.ds(ct * _TC, _TC), slice(None), pl.ds(b0, _BPW)],
                osem[h])

        def wait_out(h):
            pltpu.make_async_copy(
                obuf[h],
                out_hbm.at[pl.ds(0, _TC), slice(None), pl.ds(b0, _BPW)],
                osem[h]).wait()

        def compute(ct, h):
            src = gbuf[h]
            dst = obuf[h]
            t0 = ct * _TC
            lanes = lax.iota(jnp.int32, 16)
            for tt in range(_TC):
                pvs = [pos_v[t0 + tt, pl.ds(k * 16, 16)]
                       for k in range(_D // 16)]

                @plsc.parallel_loop(0, _BPW, unroll=8)
                def badd(b):
                    row = b * _TC + tt
                    for k in range(_D // 16):
                        sl = pl.ds(k * 16, 16)
                        src[row, sl] = src[row, sl] + pvs[k]

                r0 = lanes * _TC + tt
                r1 = r0 + 16 * _TC

                @plsc.parallel_loop(0, _D, unroll=8)
                def dloop(d):
                    cols = lax.broadcast_in_dim(d, (16,), ())
                    dst[tt, d, pl.ds(0, 16)] = plsc.load_gather(src, [r0, cols])
                    dst[tt, d, pl.ds(16, 16)] = plsc.load_gather(src, [r1, cols])

        start_gather(0, 0)
        start_gather(1, 1)

        def step(g, carry):
            for h in range(2):
                ct = 2 * g + h
                wait_gather(ct, h)

                @pl.when(g >= 1)
                def _():
                    wait_out(h)

                compute(ct, h)

                @pl.when(g < nchunk // 2 - 1)
                def _():
                    start_gather(ct + 2, h)

                start_out(ct, h)
            return carry

        lax.fori_loop(0, nchunk // 2, step, 0)
        wait_out(0)
        wait_out(1)

    return pl.kernel(
        body,
        out_type=jax.ShapeDtypeStruct((T, _D, B), jnp.float32),
        mesh=mesh,
        compiler_params=pltpu.CompilerParams(
            use_tc_tiling_on_sc=False, needs_layout_passes=False),
        scratch_types=[
            pltpu.VMEM((qtot, _QROWS), jnp.int32),
            pltpu.VMEM((T, _D), jnp.float32),
            pltpu.VMEM((rows, _D), jnp.float32),
            pltpu.VMEM((rows, _D), jnp.float32),
            pltpu.VMEM((_TC, _D, _BPW), jnp.float32),
            pltpu.VMEM((_TC, _D, _BPW), jnp.float32),
            pltpu.SemaphoreType.DMA,
            pltpu.SemaphoreType.DMA,
            pltpu.SemaphoreType.DMA,
            pltpu.SemaphoreType.DMA,
        ],
    )


def kernel(x, token_table, pos_table):
    B, T = x.shape
    _, D = token_table.shape
    xr = (x.astype(jnp.int32)
          .reshape(_NW, _BPW, T // _TC, _TC)
          .transpose(0, 2, 1, 3)
          .reshape(_NW, T // _TC * _Q, _QROWS))
    out = _build(B, T)(xr, token_table, pos_table)
    return out.transpose(2, 0, 1)


# speedup vs baseline: 1.0084x; 1.0084x over previous
import jax
import jax.numpy as jnp
from jax import lax
from jax.experimental import pallas as pl
from jax.experimental.pallas import tpu as pltpu
from jax.experimental.pallas import tpu_sc as plsc

_NC = 2
_NS = 16
_NW = _NC * _NS
_D = 64
_TC = 10
_BPW = 32
_Q = 4
_QROWS = _BPW * _TC // _Q


def _build(B, T):
    nchunk = T // _TC
    qtot = nchunk * _Q
    rows = _BPW * _TC
    mesh = plsc.VectorSubcoreMesh(
        core_axis_name="c", subcore_axis_name="s",
        num_cores=_NC, num_subcores=_NS)

    def body(idx_hbm, tok_hbm, pos_hbm, out_hbm,
             idx_v, pos_v, g0, g1, o0, o1, gs0, gs1, os0, os1):
        w = lax.axis_index("s") * _NC + lax.axis_index("c")
        b0 = w * _BPW
        pltpu.sync_copy(idx_hbm.at[w], idx_v)
        pltpu.sync_copy(pos_hbm, pos_v)

        gbuf = (g0, g1)
        obuf = (o0, o1)
        gsem = (gs0, gs1)
        osem = (os0, os1)

        def start_gather(ct, h):
            for q in range(_Q):
                pltpu.async_copy(
                    tok_hbm.at[idx_v.at[ct * _Q + q]],
                    gbuf[h].at[pl.ds(q * _QROWS, _QROWS)], gsem[h])

        def wait_gather(ct, h):
            for q in range(_Q):
                pltpu.make_async_copy(
                    tok_hbm.at[idx_v.at[ct * _Q + q]],
                    gbuf[h].at[pl.ds(q * _QROWS, _QROWS)], gsem[h]).wait()

        def start_out(ct, h):
            pltpu.async_copy(
                obuf[h],
                out_hbm.at[pl.ds(ct * _TC, _TC), slice(None), pl.ds(b0, _BPW)],
                osem[h])

        def wait_out(h):
            pltpu.make_async_copy(
                obuf[h],
                out_hbm.at[pl.ds(0, _TC), slice(None), pl.ds(b0, _BPW)],
                osem[h]).wait()

        def compute(ct, h):
            src = gbuf[h]
            dst = obuf[h]
            t0 = ct * _TC
            lanes = lax.iota(jnp.int32, 16)
            for tt in range(_TC):
                pvs = [pos_v[t0 + tt, pl.ds(k * 16, 16)]
                       for k in range(_D // 16)]

                @plsc.parallel_loop(0, _BPW, unroll=4)
                def badd(b):
                    row = b * _TC + tt
                    for k in range(_D // 16):
                        sl = pl.ds(k * 16, 16)
                        src[row, sl] = src[row, sl] + pvs[k]

                r0 = lanes * _TC + tt
                r1 = r0 + 16 * _TC

                @plsc.parallel_loop(0, _D, unroll=4)
                def dloop(d):
                    cols = lax.broadcast_in_dim(d, (16,), ())
                    dst[tt, d, pl.ds(0, 16)] = plsc.load_gather(src, [r0, cols])
                    dst[tt, d, pl.ds(16, 16)] = plsc.load_gather(src, [r1, cols])

        start_gather(0, 0)
        start_gather(1, 1)

        def step(g, carry):
            for h in range(2):
                ct = 2 * g + h
                wait_gather(ct, h)

                @pl.when(g >= 1)
                def _():
                    wait_out(h)

                compute(ct, h)

                @pl.when(g < nchunk // 2 - 1)
                def _():
                    start_gather(ct + 2, h)

                start_out(ct, h)
            return carry

        lax.fori_loop(0, nchunk // 2, step, 0)
        wait_out(0)
        wait_out(1)

    return pl.kernel(
        body,
        out_type=jax.ShapeDtypeStruct((T, _D, B), jnp.float32),
        mesh=mesh,
        compiler_params=pltpu.CompilerParams(
            use_tc_tiling_on_sc=False, needs_layout_passes=False),
        scratch_types=[
            pltpu.VMEM((qtot, _QROWS), jnp.int32),
            pltpu.VMEM((T, _D), jnp.float32),
            pltpu.VMEM((rows, _D), jnp.float32),
            pltpu.VMEM((rows, _D), jnp.float32),
            pltpu.VMEM((_TC, _D, _BPW), jnp.float32),
            pltpu.VMEM((_TC, _D, _BPW), jnp.float32),
            pltpu.SemaphoreType.DMA,
            pltpu.SemaphoreType.DMA,
            pltpu.SemaphoreType.DMA,
            pltpu.SemaphoreType.DMA,
        ],
    )


def kernel(x, token_table, pos_table):
    B, T = x.shape
    _, D = token_table.shape
    xr = (x.astype(jnp.int32)
          .reshape(_NW, _BPW, T // _TC, _TC)
          .transpose(0, 2, 1, 3)
          .reshape(_NW, T // _TC * _Q, _QROWS))
    out = _build(B, T)(xr, token_table, pos_table)
    return out.transpose(2, 0, 1)


# speedup vs baseline: 1.0185x; 1.0100x over previous
import jax
import jax.numpy as jnp
from jax import lax
from jax.experimental import pallas as pl
from jax.experimental.pallas import tpu as pltpu
from jax.experimental.pallas import tpu_sc as plsc

_NC = 2
_NS = 16
_NW = _NC * _NS
_D = 64
_TC = 10
_BPW = 32
_Q = 4
_QROWS = _BPW * _TC // _Q


def _build(B, T):
    nchunk = T // _TC
    qtot = nchunk * _Q
    rows = _BPW * _TC
    mesh = plsc.VectorSubcoreMesh(
        core_axis_name="c", subcore_axis_name="s",
        num_cores=_NC, num_subcores=_NS)

    def body(idx_hbm, tok_hbm, pos_hbm, out_hbm,
             idx_v, p0, p1, g0, g1, o0, o1, gs0, gs1, os0, os1):
        w = lax.axis_index("s") * _NC + lax.axis_index("c")
        b0 = w * _BPW
        pltpu.sync_copy(idx_hbm.at[w], idx_v)

        gbuf = (g0, g1)
        obuf = (o0, o1)
        pbuf = (p0, p1)
        gsem = (gs0, gs1)
        osem = (os0, os1)

        def start_gather(ct, h):
            for q in range(_Q):
                pltpu.async_copy(
                    tok_hbm.at[idx_v.at[ct * _Q + q]],
                    gbuf[h].at[pl.ds(q * _QROWS, _QROWS)], gsem[h])
            pltpu.async_copy(pos_hbm.at[pl.ds(ct * _TC, _TC)], pbuf[h],
                             gsem[h])

        def wait_gather(ct, h):
            for q in range(_Q):
                pltpu.make_async_copy(
                    tok_hbm.at[idx_v.at[ct * _Q + q]],
                    gbuf[h].at[pl.ds(q * _QROWS, _QROWS)], gsem[h]).wait()
            pltpu.make_async_copy(pos_hbm.at[pl.ds(0, _TC)], pbuf[h],
                                  gsem[h]).wait()

        def start_out(ct, h):
            pltpu.async_copy(
                obuf[h],
                out_hbm.at[pl.ds(ct * _TC, _TC), slice(None), pl.ds(b0, _BPW)],
                osem[h])

        def wait_out(h):
            pltpu.make_async_copy(
                obuf[h],
                out_hbm.at[pl.ds(0, _TC), slice(None), pl.ds(b0, _BPW)],
                osem[h]).wait()

        def compute(ct, h):
            src = gbuf[h]
            dst = obuf[h]
            pv = pbuf[h]
            lanes = lax.iota(jnp.int32, 16)
            for tt in range(_TC):
                r0 = lanes * _TC + tt
                r1 = r0 + 16 * _TC

                @plsc.parallel_loop(0, _D, unroll=4)
                def dloop(d):
                    cols = lax.broadcast_in_dim(d, (16,), ())
                    p = pv[tt, d, pl.ds(0, 16)]
                    dst[tt, d, pl.ds(0, 16)] = p + plsc.load_gather(
                        src, [r0, cols])
                    dst[tt, d, pl.ds(16, 16)] = p + plsc.load_gather(
                        src, [r1, cols])

        start_gather(0, 0)
        start_gather(1, 1)

        def step(g, carry):
            for h in range(2):
                ct = 2 * g + h
                wait_gather(ct, h)

                @pl.when(g >= 1)
                def _():
                    wait_out(h)

                compute(ct, h)

                @pl.when(g < nchunk // 2 - 1)
                def _():
                    start_gather(ct + 2, h)

                start_out(ct, h)
            return carry

        lax.fori_loop(0, nchunk // 2, step, 0)
        wait_out(0)
        wait_out(1)

    return pl.kernel(
        body,
        out_type=jax.ShapeDtypeStruct((T, _D, B), jnp.float32),
        mesh=mesh,
        compiler_params=pltpu.CompilerParams(
            use_tc_tiling_on_sc=False, needs_layout_passes=False),
        scratch_types=[
            pltpu.VMEM((qtot, _QROWS), jnp.int32),
            pltpu.VMEM((_TC, _D, 16), jnp.float32),
            pltpu.VMEM((_TC, _D, 16), jnp.float32),
            pltpu.VMEM((rows, _D), jnp.float32),
            pltpu.VMEM((rows, _D), jnp.float32),
            pltpu.VMEM((_TC, _D, _BPW), jnp.float32),
            pltpu.VMEM((_TC, _D, _BPW), jnp.float32),
            pltpu.SemaphoreType.DMA,
            pltpu.SemaphoreType.DMA,
            pltpu.SemaphoreType.DMA,
            pltpu.SemaphoreType.DMA,
        ],
    )


def kernel(x, token_table, pos_table):
    B, T = x.shape
    _, D = token_table.shape
    xr = (x.astype(jnp.int32)
          .reshape(_NW, _BPW, T // _TC, _TC)
          .transpose(0, 2, 1, 3)
          .reshape(_NW, T // _TC * _Q, _QROWS))
    pos16 = jnp.broadcast_to(pos_table[:, :, None], (T, D, 16))
    out = _build(B, T)(xr, token_table, pos16)
    return out.transpose(2, 0, 1)
